# Initial kernel scaffold; baseline (speedup 1.0000x reference)
import jax
import jax.numpy as jnp
from jax.experimental import pallas as pl


def kernel(x, edge_index, batch, W1, as1, ad1, b1, W2, as2, ad2, b2, Wl, bl):
    raise NotImplementedError("write your pallas kernel here")


# speedup vs baseline: 11.5561x; 11.5561x over previous
import dataclasses
import functools

import jax
import jax.numpy as jnp
from jax import lax
from jax.experimental import pallas as pl
from jax.experimental.pallas import tpu as pltpu
from jax.experimental.pallas import tpu_sc as plsc

N, D, H, HD, G = 10000, 128, 4, 256, 64
F1 = H * HD
NP = 10240
BM = 512
NBLK = NP // BM
ET = N + 320000
ETP = 331776
NC, NS = 2, 16
KB = 128
KC = 128
EPW_B = ETP // (NC * NS)
EPW_C = ETP // NS
RPT = NP // NS

def _sc_params():
    cp = pltpu.CompilerParams()
    if "needs_layout_passes" in pltpu.CompilerParams.__dataclass_fields__:
        cp = dataclasses.replace(cp, needs_layout_passes=False)
    return cp


@functools.lru_cache(maxsize=1)
def _mesh():
    return plsc.VectorSubcoreMesh(
        core_axis_name="c", subcore_axis_name="s",
        num_cores=NC, num_subcores=NS)


def _f32(*shape):
    return jax.ShapeDtypeStruct(shape, jnp.float32)


def _tc1_body(x_ref, w_ref, as_ref, ad_ref, hc_ref, asrc_ref, adst_ref,
              c_ref, mx_ref):
    i = pl.program_id(0)
    h = jnp.dot(x_ref[...], w_ref[...], preferred_element_type=jnp.float32)
    h3 = h.reshape(BM, H, HD)
    asrc = jnp.sum(h3 * as_ref[...][None], axis=-1)
    adst = jnp.sum(h3 * ad_ref[...][None], axis=-1)
    for c in range(F1 // 128):
        hc_ref[c, :, :] = h[:, c * 128:(c + 1) * 128]
    asrc_ref[...] = asrc
    adst_ref[...] = adst
    ma = jnp.max(asrc, axis=0)
    md = jnp.max(adst, axis=0)

    @pl.when(i == 0)
    def _():
        mx_ref[0, :] = ma
        mx_ref[1, :] = md

    @pl.when(i > 0)
    def _():
        mx_ref[0, :] = jnp.maximum(mx_ref[0, :], ma)
        mx_ref[1, :] = jnp.maximum(mx_ref[1, :], md)

    @pl.when(i == NBLK - 1)
    def _():
        cb = mx_ref[0, :] + mx_ref[1, :]
        cb = jnp.maximum(cb, 0.2 * cb)
        c_ref[...] = jnp.broadcast_to(cb[:, None], (H, 16))


def _tc1(x_pad, W1, as1, ad1):
    return pl.pallas_call(
        _tc1_body,
        grid=(NBLK,),
        in_specs=[
            pl.BlockSpec((BM, D), lambda i: (i, 0)),
            pl.BlockSpec((D, F1), lambda i: (0, 0)),
            pl.BlockSpec((H, HD), lambda i: (0, 0)),
            pl.BlockSpec((H, HD), lambda i: (0, 0)),
        ],
        out_specs=[
            pl.BlockSpec((F1 // 128, BM, 128), lambda i: (0, i, 0)),
            pl.BlockSpec((BM, H), lambda i: (i, 0)),
            pl.BlockSpec((BM, H), lambda i: (i, 0)),
            pl.BlockSpec((H, 16), lambda i: (0, 0)),
        ],
        out_shape=[_f32(F1 // 128, NP, 128), _f32(NP, H), _f32(NP, H),
                   _f32(H, 16)],
        scratch_shapes=[pltpu.VMEM((2, H), jnp.float32)],
    )(x_pad, W1, as1, ad1)


def _fill_rows(idxz, iota, r0):
    for t in range(8):
        idxz[0, pl.ds(16 * t, 16)] = iota + (r0 + 16 * t)


DROWS = 1280
RPD = DROWS // NS


def _sc_b_body(heads, srcf, dstf, asrcf, adstf, csp, z128,
               ex_out, den_out,
               asrc_v, adst_v, cs_v, srcb, dstb, ridxb, gbuf, exst, s_v,
               idxz, dstage, den_s):
    core = lax.axis_index("c")
    sid = lax.axis_index("s")
    w = sid * NC + core
    iota = lax.iota(jnp.int32, 16)
    pltpu.sync_copy(asrcf, asrc_v)
    pltpu.sync_copy(adstf, adst_v)
    pltpu.sync_copy(csp, cs_v)
    pltpu.sync_copy(z128, s_v)
    r0z = sid * RPD
    for t in range(8):
        idxz[0, pl.ds(16 * t, 16)] = (iota + (r0z + 16 * t)) * (t < 5)
    pltpu.sync_copy(s_v, den_s.at[idxz.at[0]])
    plsc.subcore_barrier()

    base0 = w * EPW_B
    hsel = jnp.minimum(iota, heads - 1)
    zero16 = jnp.zeros((16,), jnp.float32)

    @pl.loop(0, EPW_B, step=KB)
    def _(off):
        b = base0 + off
        pltpu.sync_copy(srcf.at[pl.ds(b, KB)], srcb.at[0])
        pltpu.sync_copy(dstf.at[pl.ds(b, KB)], dstb.at[0])
        for s in range(KB // 16):
            s16 = srcb[0, pl.ds(16 * s, 16)]
            d16 = dstb[0, pl.ds(16 * s, 16)]
            ridxb[0, pl.ds(16 * s, 16)] = d16 // 8
            gbuf[0, pl.ds(16 * s, 16)] = d16 % 8
            for h in range(heads):
                a1 = plsc.load_gather(asrc_v, [s16 * heads + h])
                a2 = plsc.load_gather(adst_v, [d16 * heads + h])
                ee = a1 + a2
                ee = jnp.where(ee < 0.0, ee * 0.2, ee)
                ex = jnp.exp(ee - cs_v[h])
                exst[h, pl.ds(16 * s, 16)] = ex

        @pl.loop(0, KB)
        def _(j):
            jb = lax.broadcast(j, (16,))
            exr = plsc.load_gather(exst, [hsel, jb])
            gsp = plsc.load_gather(gbuf, [lax.broadcast(0, (16,)), jb])
            for t in range(8):
                s_v[j, pl.ds(16 * t, 16)] = jnp.where(gsp == t, exr, zero16)

        pltpu.sync_copy(s_v, den_s.at[ridxb.at[0]], add=True)
        for h in range(heads):
            pltpu.sync_copy(exst.at[h], ex_out.at[pl.ds(h * ETP + b, KB)])

    plsc.subcore_barrier()
    pltpu.sync_copy(den_s.at[idxz.at[0]], dstage)
    pltpu.sync_copy(dstage.at[pl.ds(0, RPD)],
                    den_out.at[pl.ds(core * DROWS + r0z, RPD)])


def _sc_b(heads, srcf, dstf, asrc_flat, adst_flat, c_splat, z128):
    k = pl.kernel(
        functools.partial(_sc_b_body, heads),
        out_type=[_f32(heads * ETP), _f32(NC * DROWS, 128)],
        mesh=_mesh(),
        compiler_params=_sc_params(),
        scratch_types=[
            pltpu.VMEM((NP * heads,), jnp.float32),
            pltpu.VMEM((NP * heads,), jnp.float32),
            pltpu.VMEM((heads, 16), jnp.float32),
            pltpu.VMEM((1, KB), jnp.int32),
            pltpu.VMEM((1, KB), jnp.int32),
            pltpu.VMEM((1, KB), jnp.int32),
            pltpu.VMEM((1, KB), jnp.int32),
            pltpu.VMEM((heads, KB), jnp.float32),
            pltpu.VMEM((KB, 128), jnp.float32),
            pltpu.VMEM((1, 128), jnp.int32),
            pltpu.VMEM((128, 128), jnp.float32),
            pltpu.VMEM_SHARED((DROWS, 128), jnp.float32),
        ],
    )
    return k(srcf, dstf, asrc_flat, adst_flat, c_splat, z128)


def _sc_c_body(nch, heads, srcf, dstf, exf, hc, z128,
               a_out,
               srcb, dstb, idxb, idxz, exb, rows, a_s):
    core = lax.axis_index("c")
    sid = lax.axis_index("s")
    iota = lax.iota(jnp.int32, 16)
    cpc = nch // NC
    ebase = sid * EPW_C
    for cc in range(cpc):
        c = core * cpc + cc
        hd = c // (nch // heads)
        pltpu.sync_copy(z128, rows)
        for k in range(RPT // 128):
            _fill_rows(idxz, iota, sid * RPT + k * 128)
            pltpu.sync_copy(rows, a_s.at[idxz.at[0]])
        plsc.subcore_barrier()

        @pl.loop(0, EPW_C, step=KC)
        def _(off):
            b = ebase + off
            pltpu.sync_copy(srcf.at[pl.ds(b, KC)], srcb.at[0])
            pltpu.sync_copy(dstf.at[pl.ds(b, KC)], dstb.at[0])
            pltpu.sync_copy(exf.at[pl.ds(hd * ETP + b, KC)], exb)
            coff = c * NP
            for s in range(KC // 16):
                idxb[0, pl.ds(16 * s, 16)] = srcb[0, pl.ds(16 * s, 16)] + coff
            pltpu.sync_copy(hc.at[idxb.at[0]], rows)

            @pl.loop(0, KC)
            def _(j):
                sp = plsc.load_gather(exb, [lax.broadcast(j, (16,))])
                for t in range(8):
                    rows[j, pl.ds(16 * t, 16)] = rows[j, pl.ds(16 * t, 16)] * sp

            pltpu.sync_copy(rows, a_s.at[dstb.at[0]], add=True)

        plsc.subcore_barrier()
        for k in range(RPT // 128):
            r0 = sid * RPT + k * 128
            _fill_rows(idxz, iota, r0)
            pltpu.sync_copy(a_s.at[idxz.at[0]], rows)
            pltpu.sync_copy(rows, a_out.at[pl.ds(c * NP + r0, 128)])
        plsc.subcore_barrier()


def _sc_c(nch, heads, srcf, dstf, ex_flat, hc_flat, z128):
    k = pl.kernel(
        functools.partial(_sc_c_body, nch, heads),
        out_type=_f32(nch * NP, 128),
        mesh=_mesh(),
        compiler_params=_sc_params(),
        scratch_types=[
            pltpu.VMEM((1, KC), jnp.int32),
            pltpu.VMEM((1, KC), jnp.int32),
            pltpu.VMEM((1, KC), jnp.int32),
            pltpu.VMEM((1, 128), jnp.int32),
            pltpu.VMEM((KC,), jnp.float32),
            pltpu.VMEM((KC, 128), jnp.float32),
            pltpu.VMEM_SHARED((NP, 128), jnp.float32),
        ],
    )
    return k(srcf, dstf, ex_flat, hc_flat, z128)


def _tc2_body(a1_ref, den_ref, w2_ref, b1_ref, as2_ref, ad2_ref,
              h2c_ref, asrc2_ref, adst2_ref, c2_ref, mx_ref):
    i = pl.program_id(0)
    dent = den_ref[0, :, :H] + den_ref[1, :, :H] + 1e-16
    rows = i * BM + lax.broadcasted_iota(jnp.int32, (BM, 1), 0)
    valid = rows < N
    acc = jnp.zeros((BM, HD), jnp.float32)
    for c in range(F1 // 128):
        hd = c // ((F1 // 128) // H)
        xc = a1_ref[c, :, :] / dent[:, hd][:, None]
        xc = xc + b1_ref[c, :][None, :]
        xc = jnp.where(xc > 0.0, xc, jnp.exp(xc) - 1.0)
        xc = jnp.where(valid, xc, 0.0)
        acc = acc + jnp.dot(xc, w2_ref[c, :, :],
                            preferred_element_type=jnp.float32)
    h2c_ref[0, :, :] = acc[:, :128]
    h2c_ref[1, :, :] = acc[:, 128:]
    asrc2 = jnp.sum(acc * as2_ref[...], axis=-1)
    adst2 = jnp.sum(acc * ad2_ref[...], axis=-1)
    asrc2_ref[0, 0, :] = asrc2
    adst2_ref[0, 0, :] = adst2
    big = jnp.float32(-3.0e38)
    ma = jnp.max(jnp.where(valid[:, 0], asrc2, big))
    md = jnp.max(jnp.where(valid[:, 0], adst2, big))

    @pl.when(i == 0)
    def _():
        mx_ref[0, 0] = ma
        mx_ref[1, 0] = md

    @pl.when(i > 0)
    def _():
        mx_ref[0, 0] = jnp.maximum(mx_ref[0, 0], ma)
        mx_ref[1, 0] = jnp.maximum(mx_ref[1, 0], md)

    @pl.when(i == NBLK - 1)
    def _():
        cb = mx_ref[0, 0] + mx_ref[1, 0]
        cb = jnp.maximum(cb, 0.2 * cb)
        c2_ref[...] = jnp.full((1, 16), cb, jnp.float32)


def _tc2(a1, den1, W2c, b1c, as2, ad2):
    return pl.pallas_call(
        _tc2_body,
        grid=(NBLK,),
        in_specs=[
            pl.BlockSpec((F1 // 128, BM, 128), lambda i: (0, i, 0)),
            pl.BlockSpec((NC, BM, 16), lambda i: (0, i, 0)),
            pl.BlockSpec((F1 // 128, 128, HD), lambda i: (0, 0, 0)),
            pl.BlockSpec((F1 // 128, 128), lambda i: (0, 0)),
            pl.BlockSpec((1, HD), lambda i: (0, 0)),
            pl.BlockSpec((1, HD), lambda i: (0, 0)),
        ],
        out_specs=[
            pl.BlockSpec((2, BM, 128), lambda i: (0, i, 0)),
            pl.BlockSpec((1, 1, BM), lambda i: (i, 0, 0)),
            pl.BlockSpec((1, 1, BM), lambda i: (i, 0, 0)),
            pl.BlockSpec((1, 16), lambda i: (0, 0)),
        ],
        out_shape=[_f32(2, NP, 128), _f32(NBLK, 1, BM), _f32(NBLK, 1, BM),
                   _f32(1, 16)],
        scratch_shapes=[pltpu.SMEM((2, 1), jnp.float32)],
    )(a1, den1, W2c, b1c, as2, ad2)


def _tc3_body(a2_ref, den_ref, b2_ref, batch_ref, wl_ref, bl_ref,
              out_ref, sums_ref, cnt_ref):
    i = pl.program_id(0)
    dent = den_ref[0, :, 0] + den_ref[1, :, 0] + 1e-16
    h2 = jnp.concatenate([a2_ref[0], a2_ref[1]], axis=1) / dent[:, None]
    h2 = h2 + b2_ref[...]
    h2 = jnp.where(h2 > 0.0, h2, jnp.exp(h2) - 1.0)
    ids = batch_ref[0, 0, :]
    onehot = (ids[:, None] ==
              lax.broadcasted_iota(jnp.int32, (1, G), 1)).astype(jnp.float32)
    psum = jax.lax.dot_general(onehot, h2, (((0,), (0,)), ((), ())),
                               preferred_element_type=jnp.float32)
    pcnt = jnp.sum(onehot, axis=0)

    @pl.when(i == 0)
    def _():
        sums_ref[...] = psum
        cnt_ref[...] = pcnt[:, None]

    @pl.when(i > 0)
    def _():
        sums_ref[...] = sums_ref[...] + psum
        cnt_ref[...] = cnt_ref[...] + pcnt[:, None]

    @pl.when(i == NBLK - 1)
    def _():
        pooled = sums_ref[...] / jnp.maximum(cnt_ref[...], 1.0)
        out_ref[...] = jnp.dot(pooled, wl_ref[...],
                               preferred_element_type=jnp.float32) + bl_ref[...]


def _tc3(a2, den2, b2, batch2d, Wl, bl):
    return pl.pallas_call(
        _tc3_body,
        grid=(NBLK,),
        in_specs=[
            pl.BlockSpec((2, BM, 128), lambda i: (0, i, 0)),
            pl.BlockSpec((NC, BM, 16), lambda i: (0, i, 0)),
            pl.BlockSpec((1, HD), lambda i: (0, 0)),
            pl.BlockSpec((1, 1, BM), lambda i: (i, 0, 0)),
            pl.BlockSpec((HD, 2), lambda i: (0, 0)),
            pl.BlockSpec((1, 2), lambda i: (0, 0)),
        ],
        out_specs=pl.BlockSpec((G, 2), lambda i: (0, 0)),
        out_shape=_f32(G, 2),
        scratch_shapes=[pltpu.VMEM((G, HD), jnp.float32),
                        pltpu.VMEM((G, 1), jnp.float32)],
    )(a2, den2, b2, batch2d, Wl, bl)


def kernel(x, edge_index, batch, W1, as1, ad1, b1, W2, as2, ad2, b2, Wl, bl):
    f32 = jnp.float32
    loop = jnp.arange(N, dtype=jnp.int32)
    src = jnp.concatenate(
        [edge_index[0].astype(jnp.int32), loop,
         jnp.zeros((ETP - ET,), jnp.int32)])
    dst = jnp.concatenate(
        [edge_index[1].astype(jnp.int32), loop,
         jnp.full((ETP - ET,), N, jnp.int32)])
    x_pad = jnp.pad(x, ((0, NP - N), (0, 0)))
    batch2d = jnp.pad(batch.astype(jnp.int32), (0, NP - N),
                      constant_values=G).reshape(NBLK, 1, BM)
    z16 = jnp.zeros((128, 16), f32)
    z128 = jnp.zeros((128, 128), f32)

    h1c, asrc1, adst1, c1 = _tc1(x_pad, W1, as1, ad1)
    _DBG = 99
    if _DBG == 0:
        return (jnp.zeros((G, 2), f32) + jnp.sum(h1c) + jnp.sum(asrc1)
                + jnp.sum(adst1) + jnp.sum(c1) + jnp.sum(src) + jnp.sum(dst))
    if _DBG == 1:
        ex1, den1 = _sc_b(H, src, dst, asrc1.reshape(-1), adst1.reshape(-1),
                          c1, z128)
        return jnp.zeros((G, 2), f32) + jnp.sum(ex1) + jnp.sum(den1)
    ex1, den1 = _sc_b(H, src, dst, asrc1.reshape(-1), adst1.reshape(-1),
                      c1, z128)
    a1 = _sc_c(F1 // 128, H, src, dst, ex1, h1c.reshape(-1, 128), z128)
    if _DBG == 2:
        return jnp.zeros((G, 2), f32) + jnp.sum(a1) + jnp.sum(den1)

    W2c = W2.reshape(F1 // 128, 128, HD)
    b1c = b1.reshape(F1 // 128, 128)
    h2c, asrc2, adst2, c2 = _tc2(a1.reshape(F1 // 128, NP, 128),
                                 den1.reshape(NC, NP, 16), W2c, b1c,
                                 as2, ad2)
    ex2, den2 = _sc_b(1, src, dst, asrc2.reshape(-1), adst2.reshape(-1),
                      c2, z128)
    a2 = _sc_c(2, 1, src, dst, ex2, h2c.reshape(-1, 128), z128)

    den2r = den2.reshape(NC, NP, 16)
    return _tc3(a2.reshape(2, NP, 128), den2r,
                b2.reshape(1, HD), batch2d, Wl, bl.reshape(1, 2))


# speedup vs baseline: 14.7516x; 1.2765x over previous
import dataclasses
import functools

import jax
import jax.numpy as jnp
from jax import lax
from jax.experimental import pallas as pl
from jax.experimental.pallas import tpu as pltpu
from jax.experimental.pallas import tpu_sc as plsc

N, D, H, HD, G = 10000, 128, 4, 256, 64
F1 = H * HD
NP = 10240
BM = 512
NBLK = NP // BM
ET = N + 320000
ETP = 331776
NC, NS = 2, 16
KB = 128
KC = 128
EPW_B = ETP // (NC * NS)
EPW_C = ETP // NS
RPT = NP // NS

def _sc_params():
    cp = pltpu.CompilerParams()
    if "needs_layout_passes" in pltpu.CompilerParams.__dataclass_fields__:
        cp = dataclasses.replace(cp, needs_layout_passes=False)
    return cp


@functools.lru_cache(maxsize=1)
def _mesh():
    return plsc.VectorSubcoreMesh(
        core_axis_name="c", subcore_axis_name="s",
        num_cores=NC, num_subcores=NS)


def _f32(*shape):
    return jax.ShapeDtypeStruct(shape, jnp.float32)


def _tc1_body(x_ref, w_ref, as_ref, ad_ref, hc_ref, asrc_ref, adst_ref,
              c_ref, mx_ref):
    i = pl.program_id(0)
    h = jnp.dot(x_ref[...], w_ref[...], preferred_element_type=jnp.float32)
    h3 = h.reshape(BM, H, HD)
    asrc = jnp.sum(h3 * as_ref[...][None], axis=-1)
    adst = jnp.sum(h3 * ad_ref[...][None], axis=-1)
    for c in range(F1 // 128):
        hc_ref[c, :, :] = h[:, c * 128:(c + 1) * 128]
    asrc_ref[...] = asrc
    adst_ref[...] = adst
    ma = jnp.max(asrc, axis=0)
    md = jnp.max(adst, axis=0)

    @pl.when(i == 0)
    def _():
        mx_ref[0, :] = ma
        mx_ref[1, :] = md

    @pl.when(i > 0)
    def _():
        mx_ref[0, :] = jnp.maximum(mx_ref[0, :], ma)
        mx_ref[1, :] = jnp.maximum(mx_ref[1, :], md)

    @pl.when(i == NBLK - 1)
    def _():
        cb = mx_ref[0, :] + mx_ref[1, :]
        cb = jnp.maximum(cb, 0.2 * cb)
        c_ref[...] = jnp.broadcast_to(cb[:, None], (H, 16))


def _tc1(x_pad, W1, as1, ad1):
    return pl.pallas_call(
        _tc1_body,
        grid=(NBLK,),
        in_specs=[
            pl.BlockSpec((BM, D), lambda i: (i, 0)),
            pl.BlockSpec((D, F1), lambda i: (0, 0)),
            pl.BlockSpec((H, HD), lambda i: (0, 0)),
            pl.BlockSpec((H, HD), lambda i: (0, 0)),
        ],
        out_specs=[
            pl.BlockSpec((F1 // 128, BM, 128), lambda i: (0, i, 0)),
            pl.BlockSpec((BM, H), lambda i: (i, 0)),
            pl.BlockSpec((BM, H), lambda i: (i, 0)),
            pl.BlockSpec((H, 16), lambda i: (0, 0)),
        ],
        out_shape=[_f32(F1 // 128, NP, 128), _f32(NP, H), _f32(NP, H),
                   _f32(H, 16)],
        scratch_shapes=[pltpu.VMEM((2, H), jnp.float32)],
    )(x_pad, W1, as1, ad1)


def _fill_rows(idxz, iota, r0):
    for t in range(8):
        idxz[0, pl.ds(16 * t, 16)] = iota + (r0 + 16 * t)


DROWS = 1280
RPD = DROWS // NS


def _sc_b_body(heads, srcf, dstf, asrcf, adstf, csp, z128,
               ex_out, den_out,
               asrc_v, adst_v, cs_v, srcb, dstb, ridxb, gbuf, exst, s_v,
               idxz, dstage, den_s):
    core = lax.axis_index("c")
    sid = lax.axis_index("s")
    w = sid * NC + core
    iota = lax.iota(jnp.int32, 16)
    pltpu.sync_copy(asrcf, asrc_v)
    pltpu.sync_copy(adstf, adst_v)
    pltpu.sync_copy(csp, cs_v)
    pltpu.sync_copy(z128, s_v)
    r0z = sid * RPD
    for t in range(8):
        idxz[0, pl.ds(16 * t, 16)] = (iota + (r0z + 16 * t)) * (t < 5)
    pltpu.sync_copy(s_v, den_s.at[idxz.at[0]])
    plsc.subcore_barrier()

    base0 = w * EPW_B
    hsel = jnp.minimum(iota, heads - 1)
    zero16 = jnp.zeros((16,), jnp.float32)

    @pl.loop(0, EPW_B, step=KB)
    def _(off):
        b = base0 + off
        pltpu.sync_copy(srcf.at[pl.ds(b, KB)], srcb.at[0])
        pltpu.sync_copy(dstf.at[pl.ds(b, KB)], dstb.at[0])
        for s in range(KB // 16):
            s16 = srcb[0, pl.ds(16 * s, 16)]
            d16 = dstb[0, pl.ds(16 * s, 16)]
            ridxb[0, pl.ds(16 * s, 16)] = d16 // 8
            gbuf[0, pl.ds(16 * s, 16)] = d16 % 8
            for h in range(heads):
                a1 = plsc.load_gather(asrc_v, [s16 * heads + h])
                a2 = plsc.load_gather(adst_v, [d16 * heads + h])
                ee = a1 + a2
                ee = jnp.where(ee < 0.0, ee * 0.2, ee)
                ex = jnp.exp(ee - cs_v[h])
                exst[h, pl.ds(16 * s, 16)] = ex

        @pl.loop(0, KB)
        def _(j):
            jb = lax.broadcast(j, (16,))
            exr = plsc.load_gather(exst, [hsel, jb])
            gsp = plsc.load_gather(gbuf, [lax.broadcast(0, (16,)), jb])
            for t in range(8):
                s_v[j, pl.ds(16 * t, 16)] = jnp.where(gsp == t, exr, zero16)

        pltpu.sync_copy(s_v, den_s.at[ridxb.at[0]], add=True)
        for h in range(heads):
            pltpu.sync_copy(exst.at[h], ex_out.at[pl.ds(h * ETP + b, KB)])

    plsc.subcore_barrier()
    pltpu.sync_copy(den_s.at[idxz.at[0]], dstage)
    pltpu.sync_copy(dstage.at[pl.ds(0, RPD)],
                    den_out.at[pl.ds(core * DROWS + r0z, RPD)])


def _sc_b(heads, srcf, dstf, asrc_flat, adst_flat, c_splat, z128):
    k = pl.kernel(
        functools.partial(_sc_b_body, heads),
        out_type=[_f32(heads * ETP), _f32(NC * DROWS, 128)],
        mesh=_mesh(),
        compiler_params=_sc_params(),
        scratch_types=[
            pltpu.VMEM((NP * heads,), jnp.float32),
            pltpu.VMEM((NP * heads,), jnp.float32),
            pltpu.VMEM((heads, 16), jnp.float32),
            pltpu.VMEM((1, KB), jnp.int32),
            pltpu.VMEM((1, KB), jnp.int32),
            pltpu.VMEM((1, KB), jnp.int32),
            pltpu.VMEM((1, KB), jnp.int32),
            pltpu.VMEM((heads, KB), jnp.float32),
            pltpu.VMEM((KB, 128), jnp.float32),
            pltpu.VMEM((1, 128), jnp.int32),
            pltpu.VMEM((128, 128), jnp.float32),
            pltpu.VMEM_SHARED((DROWS, 128), jnp.float32),
        ],
    )
    return k(srcf, dstf, asrc_flat, adst_flat, c_splat, z128)


def _sc_c_body(nch, heads, srcf, dstf, exf, hc, z128,
               a_out,
               srcb, dstb2, idxb2, idxz, exb, rows_a, rows_b, sem, a_s):
    core = lax.axis_index("c")
    sid = lax.axis_index("s")
    iota = lax.iota(jnp.int32, 16)
    cpc = nch // NC
    ebase = sid * EPW_C
    KC2 = 2 * KC
    for cc in range(cpc):
        c = core * cpc + cc
        hd = c // (nch // heads)
        pltpu.sync_copy(z128, rows_a)
        for k in range(RPT // 128):
            _fill_rows(idxz, iota, sid * RPT + k * 128)
            pltpu.sync_copy(rows_a, a_s.at[idxz.at[0]])
        plsc.subcore_barrier()

        @pl.loop(0, EPW_C, step=KC2)
        def _(off):
            b = ebase + off
            l1 = pltpu.async_copy(srcf.at[pl.ds(b, KC)], srcb.at[0], sem)
            l2 = pltpu.async_copy(srcf.at[pl.ds(b + KC, KC)], srcb.at[1], sem)
            l3 = pltpu.async_copy(dstf.at[pl.ds(b, KC)], dstb2.at[0], sem)
            l4 = pltpu.async_copy(dstf.at[pl.ds(b + KC, KC)], dstb2.at[1], sem)
            l5 = pltpu.async_copy(exf.at[pl.ds(hd * ETP + b, KC2)], exb, sem)
            l1.wait()
            l2.wait()
            l3.wait()
            l4.wait()
            l5.wait()
            coff = c * NP
            for g in range(2):
                for s in range(KC // 16):
                    idxb2[g, pl.ds(16 * s, 16)] = (
                        srcb[g, pl.ds(16 * s, 16)] + coff)
            g1 = pltpu.async_copy(hc.at[idxb2.at[0]], rows_a, sem)
            g2 = pltpu.async_copy(hc.at[idxb2.at[1]], rows_b, sem)
            g1.wait()
            g2.wait()
            for g, buf in enumerate((rows_a, rows_b)):
                @pl.loop(0, KC)
                def _(j):
                    sp = plsc.load_gather(
                        exb, [lax.broadcast(j + g * KC, (16,))])
                    for t in range(8):
                        buf[j, pl.ds(16 * t, 16)] = (
                            buf[j, pl.ds(16 * t, 16)] * sp)
            s1 = pltpu.async_copy(rows_a, a_s.at[dstb2.at[0]], sem, add=True)
            s2 = pltpu.async_copy(rows_b, a_s.at[dstb2.at[1]], sem, add=True)
            s1.wait()
            s2.wait()

        plsc.subcore_barrier()
        for k in range(RPT // 128):
            r0 = sid * RPT + k * 128
            _fill_rows(idxz, iota, r0)
            pltpu.sync_copy(a_s.at[idxz.at[0]], rows_a)
            pltpu.sync_copy(rows_a, a_out.at[pl.ds(c * NP + r0, 128)])
        plsc.subcore_barrier()


def _sc_c(nch, heads, srcf, dstf, ex_flat, hc_flat, z128):
    k = pl.kernel(
        functools.partial(_sc_c_body, nch, heads),
        out_type=_f32(nch * NP, 128),
        mesh=_mesh(),
        compiler_params=_sc_params(),
        scratch_types=[
            pltpu.VMEM((2, KC), jnp.int32),
            pltpu.VMEM((2, KC), jnp.int32),
            pltpu.VMEM((2, KC), jnp.int32),
            pltpu.VMEM((1, 128), jnp.int32),
            pltpu.VMEM((2 * KC,), jnp.float32),
            pltpu.VMEM((KC, 128), jnp.float32),
            pltpu.VMEM((KC, 128), jnp.float32),
            pltpu.SemaphoreType.DMA,
            pltpu.VMEM_SHARED((NP, 128), jnp.float32),
        ],
    )
    return k(srcf, dstf, ex_flat, hc_flat, z128)


def _tc2_body(a1_ref, den_ref, w2_ref, b1_ref, as2_ref, ad2_ref,
              h2c_ref, asrc2_ref, adst2_ref, c2_ref, mx_ref):
    i = pl.program_id(0)
    dent = den_ref[0, :, :H] + den_ref[1, :, :H] + 1e-16
    rows = i * BM + lax.broadcasted_iota(jnp.int32, (BM, 1), 0)
    valid = rows < N
    acc = jnp.zeros((BM, HD), jnp.float32)
    for c in range(F1 // 128):
        hd = c // ((F1 // 128) // H)
        xc = a1_ref[c, :, :] / dent[:, hd][:, None]
        xc = xc + b1_ref[c, :][None, :]
        xc = jnp.where(xc > 0.0, xc, jnp.exp(xc) - 1.0)
        xc = jnp.where(valid, xc, 0.0)
        acc = acc + jnp.dot(xc, w2_ref[c, :, :],
                            preferred_element_type=jnp.float32)
    h2c_ref[0, :, :] = acc[:, :128]
    h2c_ref[1, :, :] = acc[:, 128:]
    asrc2 = jnp.sum(acc * as2_ref[...], axis=-1)
    adst2 = jnp.sum(acc * ad2_ref[...], axis=-1)
    asrc2_ref[0, 0, :] = asrc2
    adst2_ref[0, 0, :] = adst2
    big = jnp.float32(-3.0e38)
    ma = jnp.max(jnp.where(valid[:, 0], asrc2, big))
    md = jnp.max(jnp.where(valid[:, 0], adst2, big))

    @pl.when(i == 0)
    def _():
        mx_ref[0, 0] = ma
        mx_ref[1, 0] = md

    @pl.when(i > 0)
    def _():
        mx_ref[0, 0] = jnp.maximum(mx_ref[0, 0], ma)
        mx_ref[1, 0] = jnp.maximum(mx_ref[1, 0], md)

    @pl.when(i == NBLK - 1)
    def _():
        cb = mx_ref[0, 0] + mx_ref[1, 0]
        cb = jnp.maximum(cb, 0.2 * cb)
        c2_ref[...] = jnp.full((1, 16), cb, jnp.float32)


def _tc2(a1, den1, W2c, b1c, as2, ad2):
    return pl.pallas_call(
        _tc2_body,
        grid=(NBLK,),
        in_specs=[
            pl.BlockSpec((F1 // 128, BM, 128), lambda i: (0, i, 0)),
            pl.BlockSpec((NC, BM, 16), lambda i: (0, i, 0)),
            pl.BlockSpec((F1 // 128, 128, HD), lambda i: (0, 0, 0)),
            pl.BlockSpec((F1 // 128, 128), lambda i: (0, 0)),
            pl.BlockSpec((1, HD), lambda i: (0, 0)),
            pl.BlockSpec((1, HD), lambda i: (0, 0)),
        ],
        out_specs=[
            pl.BlockSpec((2, BM, 128), lambda i: (0, i, 0)),
            pl.BlockSpec((1, 1, BM), lambda i: (i, 0, 0)),
            pl.BlockSpec((1, 1, BM), lambda i: (i, 0, 0)),
            pl.BlockSpec((1, 16), lambda i: (0, 0)),
        ],
        out_shape=[_f32(2, NP, 128), _f32(NBLK, 1, BM), _f32(NBLK, 1, BM),
                   _f32(1, 16)],
        scratch_shapes=[pltpu.SMEM((2, 1), jnp.float32)],
    )(a1, den1, W2c, b1c, as2, ad2)


def _tc3_body(a2_ref, den_ref, b2_ref, batch_ref, wl_ref, bl_ref,
              out_ref, sums_ref, cnt_ref):
    i = pl.program_id(0)
    dent = den_ref[0, :, 0] + den_ref[1, :, 0] + 1e-16
    h2 = jnp.concatenate([a2_ref[0], a2_ref[1]], axis=1) / dent[:, None]
    h2 = h2 + b2_ref[...]
    h2 = jnp.where(h2 > 0.0, h2, jnp.exp(h2) - 1.0)
    ids = batch_ref[0, 0, :]
    onehot = (ids[:, None] ==
              lax.broadcasted_iota(jnp.int32, (1, G), 1)).astype(jnp.float32)
    psum = jax.lax.dot_general(onehot, h2, (((0,), (0,)), ((), ())),
                               preferred_element_type=jnp.float32)
    pcnt = jnp.sum(onehot, axis=0)

    @pl.when(i == 0)
    def _():
        sums_ref[...] = psum
        cnt_ref[...] = pcnt[:, None]

    @pl.when(i > 0)
    def _():
        sums_ref[...] = sums_ref[...] + psum
        cnt_ref[...] = cnt_ref[...] + pcnt[:, None]

    @pl.when(i == NBLK - 1)
    def _():
        pooled = sums_ref[...] / jnp.maximum(cnt_ref[...], 1.0)
        out_ref[...] = jnp.dot(pooled, wl_ref[...],
                               preferred_element_type=jnp.float32) + bl_ref[...]


def _tc3(a2, den2, b2, batch2d, Wl, bl):
    return pl.pallas_call(
        _tc3_body,
        grid=(NBLK,),
        in_specs=[
            pl.BlockSpec((2, BM, 128), lambda i: (0, i, 0)),
            pl.BlockSpec((NC, BM, 16), lambda i: (0, i, 0)),
            pl.BlockSpec((1, HD), lambda i: (0, 0)),
            pl.BlockSpec((1, 1, BM), lambda i: (i, 0, 0)),
            pl.BlockSpec((HD, 2), lambda i: (0, 0)),
            pl.BlockSpec((1, 2), lambda i: (0, 0)),
        ],
        out_specs=pl.BlockSpec((G, 2), lambda i: (0, 0)),
        out_shape=_f32(G, 2),
        scratch_shapes=[pltpu.VMEM((G, HD), jnp.float32),
                        pltpu.VMEM((G, 1), jnp.float32)],
    )(a2, den2, b2, batch2d, Wl, bl)


def kernel(x, edge_index, batch, W1, as1, ad1, b1, W2, as2, ad2, b2, Wl, bl):
    f32 = jnp.float32
    loop = jnp.arange(N, dtype=jnp.int32)
    src = jnp.concatenate(
        [edge_index[0].astype(jnp.int32), loop,
         jnp.zeros((ETP - ET,), jnp.int32)])
    dst = jnp.concatenate(
        [edge_index[1].astype(jnp.int32), loop,
         jnp.full((ETP - ET,), N, jnp.int32)])
    x_pad = jnp.pad(x, ((0, NP - N), (0, 0)))
    batch2d = jnp.pad(batch.astype(jnp.int32), (0, NP - N),
                      constant_values=G).reshape(NBLK, 1, BM)
    z16 = jnp.zeros((128, 16), f32)
    z128 = jnp.zeros((128, 128), f32)

    h1c, asrc1, adst1, c1 = _tc1(x_pad, W1, as1, ad1)
    _DBG = 99
    if _DBG == 0:
        return (jnp.zeros((G, 2), f32) + jnp.sum(h1c) + jnp.sum(asrc1)
                + jnp.sum(adst1) + jnp.sum(c1) + jnp.sum(src) + jnp.sum(dst))
    if _DBG == 1:
        ex1, den1 = _sc_b(H, src, dst, asrc1.reshape(-1), adst1.reshape(-1),
                          c1, z128)
        return jnp.zeros((G, 2), f32) + jnp.sum(ex1) + jnp.sum(den1)
    ex1, den1 = _sc_b(H, src, dst, asrc1.reshape(-1), adst1.reshape(-1),
                      c1, z128)
    a1 = _sc_c(F1 // 128, H, src, dst, ex1, h1c.reshape(-1, 128), z128)
    if _DBG == 2:
        return jnp.zeros((G, 2), f32) + jnp.sum(a1) + jnp.sum(den1)

    W2c = W2.reshape(F1 // 128, 128, HD)
    b1c = b1.reshape(F1 // 128, 128)
    h2c, asrc2, adst2, c2 = _tc2(a1.reshape(F1 // 128, NP, 128),
                                 den1.reshape(NC, NP, 16), W2c, b1c,
                                 as2, ad2)
    ex2, den2 = _sc_b(1, src, dst, asrc2.reshape(-1), adst2.reshape(-1),
                      c2, z128)
    a2 = _sc_c(2, 1, src, dst, ex2, h2c.reshape(-1, 128), z128)

    den2r = den2.reshape(NC, NP, 16)
    return _tc3(a2.reshape(2, NP, 128), den2r,
                b2.reshape(1, HD), batch2d, Wl, bl.reshape(1, 2))


# speedup vs baseline: 14.7836x; 1.0022x over previous
import dataclasses
import functools

import jax
import jax.numpy as jnp
from jax import lax
from jax.experimental import pallas as pl
from jax.experimental.pallas import tpu as pltpu
from jax.experimental.pallas import tpu_sc as plsc

N, D, H, HD, G = 10000, 128, 4, 256, 64
F1 = H * HD
NP = 10240
BM = 512
NBLK = NP // BM
ET = N + 320000
ETP = 331776
NC, NS = 2, 16
KB = 128
KC = 128
EPW_B = ETP // (NC * NS)
EPW_C = ETP // NS
RPT = NP // NS

def _sc_params():
    cp = pltpu.CompilerParams()
    if "needs_layout_passes" in pltpu.CompilerParams.__dataclass_fields__:
        cp = dataclasses.replace(cp, needs_layout_passes=False)
    return cp


@functools.lru_cache(maxsize=1)
def _mesh():
    return plsc.VectorSubcoreMesh(
        core_axis_name="c", subcore_axis_name="s",
        num_cores=NC, num_subcores=NS)


def _f32(*shape):
    return jax.ShapeDtypeStruct(shape, jnp.float32)


def _tc1_body(x_ref, w_ref, as_ref, ad_ref, hc_ref, asrc_ref, adst_ref,
              c_ref, mx_ref):
    i = pl.program_id(0)
    h = jnp.dot(x_ref[...], w_ref[...], preferred_element_type=jnp.float32)
    h3 = h.reshape(BM, H, HD)
    asrc = jnp.sum(h3 * as_ref[...][None], axis=-1)
    adst = jnp.sum(h3 * ad_ref[...][None], axis=-1)
    for c in range(F1 // 128):
        hc_ref[c, :, :] = h[:, c * 128:(c + 1) * 128]
    asrc_ref[...] = asrc
    adst_ref[...] = adst
    ma = jnp.max(asrc, axis=0)
    md = jnp.max(adst, axis=0)

    @pl.when(i == 0)
    def _():
        mx_ref[0, :] = ma
        mx_ref[1, :] = md

    @pl.when(i > 0)
    def _():
        mx_ref[0, :] = jnp.maximum(mx_ref[0, :], ma)
        mx_ref[1, :] = jnp.maximum(mx_ref[1, :], md)

    @pl.when(i == NBLK - 1)
    def _():
        cb = mx_ref[0, :] + mx_ref[1, :]
        cb = jnp.maximum(cb, 0.2 * cb)
        c_ref[...] = jnp.broadcast_to(cb[:, None], (H, 16))


def _tc1(x_pad, W1, as1, ad1):
    return pl.pallas_call(
        _tc1_body,
        grid=(NBLK,),
        in_specs=[
            pl.BlockSpec((BM, D), lambda i: (i, 0)),
            pl.BlockSpec((D, F1), lambda i: (0, 0)),
            pl.BlockSpec((H, HD), lambda i: (0, 0)),
            pl.BlockSpec((H, HD), lambda i: (0, 0)),
        ],
        out_specs=[
            pl.BlockSpec((F1 // 128, BM, 128), lambda i: (0, i, 0)),
            pl.BlockSpec((BM, H), lambda i: (i, 0)),
            pl.BlockSpec((BM, H), lambda i: (i, 0)),
            pl.BlockSpec((H, 16), lambda i: (0, 0)),
        ],
        out_shape=[_f32(F1 // 128, NP, 128), _f32(NP, H), _f32(NP, H),
                   _f32(H, 16)],
        scratch_shapes=[pltpu.VMEM((2, H), jnp.float32)],
    )(x_pad, W1, as1, ad1)


def _fill_rows(idxz, iota, r0):
    for t in range(8):
        idxz[0, pl.ds(16 * t, 16)] = iota + (r0 + 16 * t)


DROWS = 1280
RPD = DROWS // NS


def _sc_b_body(heads, srcf, dstf, asrcf, adstf, csp, z128,
               ex_out, den_out,
               asrc_v, adst_v, cs_v, srcb, dstb, ridxb, gbuf, exst, s_v,
               idxz, dstage, den_s):
    core = lax.axis_index("c")
    sid = lax.axis_index("s")
    w = sid * NC + core
    iota = lax.iota(jnp.int32, 16)
    pltpu.sync_copy(asrcf, asrc_v)
    pltpu.sync_copy(adstf, adst_v)
    pltpu.sync_copy(csp, cs_v)
    pltpu.sync_copy(z128, s_v)
    r0z = sid * RPD
    for t in range(8):
        idxz[0, pl.ds(16 * t, 16)] = (iota + (r0z + 16 * t)) * (t < 5)
    pltpu.sync_copy(s_v, den_s.at[idxz.at[0]])
    plsc.subcore_barrier()

    base0 = w * EPW_B
    hsel = jnp.minimum(iota, heads - 1)
    zero16 = jnp.zeros((16,), jnp.float32)

    @pl.loop(0, EPW_B, step=KB)
    def _(off):
        b = base0 + off
        pltpu.sync_copy(srcf.at[pl.ds(b, KB)], srcb.at[0])
        pltpu.sync_copy(dstf.at[pl.ds(b, KB)], dstb.at[0])
        for s in range(KB // 16):
            s16 = srcb[0, pl.ds(16 * s, 16)]
            d16 = dstb[0, pl.ds(16 * s, 16)]
            ridxb[0, pl.ds(16 * s, 16)] = d16 // 8
            gbuf[0, pl.ds(16 * s, 16)] = d16 % 8
            for h in range(heads):
                a1 = plsc.load_gather(asrc_v, [s16 * heads + h])
                a2 = plsc.load_gather(adst_v, [d16 * heads + h])
                ee = a1 + a2
                ee = jnp.where(ee < 0.0, ee * 0.2, ee)
                ex = jnp.exp(ee - cs_v[h])
                exst[h, pl.ds(16 * s, 16)] = ex

        @pl.loop(0, KB)
        def _(j):
            jb = lax.broadcast(j, (16,))
            exr = plsc.load_gather(exst, [hsel, jb])
            gsp = plsc.load_gather(gbuf, [lax.broadcast(0, (16,)), jb])
            for t in range(8):
                s_v[j, pl.ds(16 * t, 16)] = jnp.where(gsp == t, exr, zero16)

        pltpu.sync_copy(s_v, den_s.at[ridxb.at[0]], add=True)
        for h in range(heads):
            pltpu.sync_copy(exst.at[h], ex_out.at[pl.ds(h * ETP + b, KB)])

    plsc.subcore_barrier()
    pltpu.sync_copy(den_s.at[idxz.at[0]], dstage)
    pltpu.sync_copy(dstage.at[pl.ds(0, RPD)],
                    den_out.at[pl.ds(core * DROWS + r0z, RPD)])


def _sc_b(heads, srcf, dstf, asrc_flat, adst_flat, c_splat, z128):
    k = pl.kernel(
        functools.partial(_sc_b_body, heads),
        out_type=[_f32(heads * ETP), _f32(NC * DROWS, 128)],
        mesh=_mesh(),
        compiler_params=_sc_params(),
        scratch_types=[
            pltpu.VMEM((NP * heads,), jnp.float32),
            pltpu.VMEM((NP * heads,), jnp.float32),
            pltpu.VMEM((heads, 16), jnp.float32),
            pltpu.VMEM((1, KB), jnp.int32),
            pltpu.VMEM((1, KB), jnp.int32),
            pltpu.VMEM((1, KB), jnp.int32),
            pltpu.VMEM((1, KB), jnp.int32),
            pltpu.VMEM((heads, KB), jnp.float32),
            pltpu.VMEM((KB, 128), jnp.float32),
            pltpu.VMEM((1, 128), jnp.int32),
            pltpu.VMEM((128, 128), jnp.float32),
            pltpu.VMEM_SHARED((DROWS, 128), jnp.float32),
        ],
    )
    return k(srcf, dstf, asrc_flat, adst_flat, c_splat, z128)


def _sc_c_body(nch, heads, srcf, dstf, exf, hc, z128,
               a_out,
               srcb, dstb2, idxb2, idxz, exb, rows_a, rows_b, sem, a_s):
    core = lax.axis_index("c")
    sid = lax.axis_index("s")
    iota = lax.iota(jnp.int32, 16)
    cpc = nch // NC
    ebase = sid * EPW_C
    KC2 = 2 * KC
    for cc in range(cpc):
        c = core * cpc + cc
        hd = c // (nch // heads)
        pltpu.sync_copy(z128, rows_a)
        for k in range(RPT // 128):
            _fill_rows(idxz, iota, sid * RPT + k * 128)
            pltpu.sync_copy(rows_a, a_s.at[idxz.at[0]])
        plsc.subcore_barrier()

        @pl.loop(0, EPW_C, step=KC2)
        def _(off):
            b = ebase + off
            l1 = pltpu.async_copy(srcf.at[pl.ds(b, KC)], srcb.at[0], sem)
            l2 = pltpu.async_copy(srcf.at[pl.ds(b + KC, KC)], srcb.at[1], sem)
            l3 = pltpu.async_copy(dstf.at[pl.ds(b, KC)], dstb2.at[0], sem)
            l4 = pltpu.async_copy(dstf.at[pl.ds(b + KC, KC)], dstb2.at[1], sem)
            l5 = pltpu.async_copy(exf.at[pl.ds(hd * ETP + b, KC2)], exb, sem)
            l1.wait()
            l2.wait()
            l3.wait()
            l4.wait()
            l5.wait()
            coff = c * NP
            for g in range(2):
                for s in range(KC // 16):
                    idxb2[g, pl.ds(16 * s, 16)] = (
                        srcb[g, pl.ds(16 * s, 16)] + coff)
            g1 = pltpu.async_copy(hc.at[idxb2.at[0]], rows_a, sem)
            g2 = pltpu.async_copy(hc.at[idxb2.at[1]], rows_b, sem)
            g1.wait()
            g2.wait()
            for g, buf in enumerate((rows_a, rows_b)):
                @pl.loop(0, KC)
                def _(j):
                    sp = plsc.load_gather(
                        exb, [lax.broadcast(j + g * KC, (16,))])
                    for t in range(8):
                        buf[j, pl.ds(16 * t, 16)] = (
                            buf[j, pl.ds(16 * t, 16)] * sp)
            s1 = pltpu.async_copy(rows_a, a_s.at[dstb2.at[0]], sem, add=True)
            s2 = pltpu.async_copy(rows_b, a_s.at[dstb2.at[1]], sem, add=True)
            s1.wait()
            s2.wait()

        plsc.subcore_barrier()
        for k in range(RPT // 128):
            r0 = sid * RPT + k * 128
            _fill_rows(idxz, iota, r0)
            pltpu.sync_copy(a_s.at[idxz.at[0]], rows_a)
            pltpu.sync_copy(rows_a, a_out.at[pl.ds(c * NP + r0, 128)])
        plsc.subcore_barrier()


def _sc_c(nch, heads, srcf, dstf, ex_flat, hc_flat, z128):
    k = pl.kernel(
        functools.partial(_sc_c_body, nch, heads),
        out_type=_f32(nch * NP, 128),
        mesh=_mesh(),
        compiler_params=_sc_params(),
        scratch_types=[
            pltpu.VMEM((2, KC), jnp.int32),
            pltpu.VMEM((2, KC), jnp.int32),
            pltpu.VMEM((2, KC), jnp.int32),
            pltpu.VMEM((1, 128), jnp.int32),
            pltpu.VMEM((2 * KC,), jnp.float32),
            pltpu.VMEM((KC, 128), jnp.float32),
            pltpu.VMEM((KC, 128), jnp.float32),
            pltpu.SemaphoreType.DMA,
            pltpu.VMEM_SHARED((NP, 128), jnp.float32),
        ],
    )
    return k(srcf, dstf, ex_flat, hc_flat, z128)


def _tc2_body(a1_ref, den_ref, w2_ref, b1_ref, as2_ref, ad2_ref,
              h2c_ref, asrc2_ref, adst2_ref, c2_ref, mx_ref):
    i = pl.program_id(0)
    dent = den_ref[0, :, :H] + den_ref[1, :, :H] + 1e-16
    rows = i * BM + lax.broadcasted_iota(jnp.int32, (BM, 1), 0)
    valid = rows < N
    acc = jnp.zeros((BM, HD), jnp.float32)
    for c in range(F1 // 128):
        hd = c // ((F1 // 128) // H)
        xc = a1_ref[c, :, :] / dent[:, hd][:, None]
        xc = xc + b1_ref[c, :][None, :]
        xc = jnp.where(xc > 0.0, xc, jnp.exp(xc) - 1.0)
        xc = jnp.where(valid, xc, 0.0)
        acc = acc + jnp.dot(xc, w2_ref[c, :, :],
                            preferred_element_type=jnp.float32)
    h2c_ref[0, :, :] = acc[:, :128]
    h2c_ref[1, :, :] = acc[:, 128:]
    asrc2 = jnp.sum(acc * as2_ref[...], axis=-1)
    adst2 = jnp.sum(acc * ad2_ref[...], axis=-1)
    asrc2_ref[0, 0, :] = asrc2
    adst2_ref[0, 0, :] = adst2
    big = jnp.float32(-3.0e38)
    ma = jnp.max(jnp.where(valid[:, 0], asrc2, big))
    md = jnp.max(jnp.where(valid[:, 0], adst2, big))

    @pl.when(i == 0)
    def _():
        mx_ref[0, 0] = ma
        mx_ref[1, 0] = md

    @pl.when(i > 0)
    def _():
        mx_ref[0, 0] = jnp.maximum(mx_ref[0, 0], ma)
        mx_ref[1, 0] = jnp.maximum(mx_ref[1, 0], md)

    @pl.when(i == NBLK - 1)
    def _():
        cb = mx_ref[0, 0] + mx_ref[1, 0]
        cb = jnp.maximum(cb, 0.2 * cb)
        c2_ref[...] = jnp.full((1, 16), cb, jnp.float32)


def _tc2(a1, den1, W2c, b1c, as2, ad2):
    return pl.pallas_call(
        _tc2_body,
        grid=(NBLK,),
        in_specs=[
            pl.BlockSpec((F1 // 128, BM, 128), lambda i: (0, i, 0)),
            pl.BlockSpec((NC, BM, 16), lambda i: (0, i, 0)),
            pl.BlockSpec((F1 // 128, 128, HD), lambda i: (0, 0, 0)),
            pl.BlockSpec((F1 // 128, 128), lambda i: (0, 0)),
            pl.BlockSpec((1, HD), lambda i: (0, 0)),
            pl.BlockSpec((1, HD), lambda i: (0, 0)),
        ],
        out_specs=[
            pl.BlockSpec((2, BM, 128), lambda i: (0, i, 0)),
            pl.BlockSpec((1, 1, BM), lambda i: (i, 0, 0)),
            pl.BlockSpec((1, 1, BM), lambda i: (i, 0, 0)),
            pl.BlockSpec((1, 16), lambda i: (0, 0)),
        ],
        out_shape=[_f32(2, NP, 128), _f32(NBLK, 1, BM), _f32(NBLK, 1, BM),
                   _f32(1, 16)],
        scratch_shapes=[pltpu.SMEM((2, 1), jnp.float32)],
    )(a1, den1, W2c, b1c, as2, ad2)


def _tc3_body(a2_ref, den_ref, b2_ref, batch_ref, wl_ref, bl_ref,
              out_ref, sums_ref, cnt_ref):
    i = pl.program_id(0)
    dent = den_ref[0, :, 0] + den_ref[1, :, 0] + 1e-16
    h2 = jnp.concatenate([a2_ref[0], a2_ref[1]], axis=1) / dent[:, None]
    h2 = h2 + b2_ref[...]
    h2 = jnp.where(h2 > 0.0, h2, jnp.exp(h2) - 1.0)
    ids = batch_ref[0, 0, :]
    onehot = (ids[:, None] ==
              lax.broadcasted_iota(jnp.int32, (1, G), 1)).astype(jnp.float32)
    psum = jax.lax.dot_general(onehot, h2, (((0,), (0,)), ((), ())),
                               preferred_element_type=jnp.float32)
    pcnt = jnp.sum(onehot, axis=0)

    @pl.when(i == 0)
    def _():
        sums_ref[...] = psum
        cnt_ref[...] = pcnt[:, None]

    @pl.when(i > 0)
    def _():
        sums_ref[...] = sums_ref[...] + psum
        cnt_ref[...] = cnt_ref[...] + pcnt[:, None]

    @pl.when(i == NBLK - 1)
    def _():
        pooled = sums_ref[...] / jnp.maximum(cnt_ref[...], 1.0)
        out_ref[...] = jnp.dot(pooled, wl_ref[...],
                               preferred_element_type=jnp.float32) + bl_ref[...]


def _tc3(a2, den2, b2, batch2d, Wl, bl):
    return pl.pallas_call(
        _tc3_body,
        grid=(NBLK,),
        in_specs=[
            pl.BlockSpec((2, BM, 128), lambda i: (0, i, 0)),
            pl.BlockSpec((NC, BM, 16), lambda i: (0, i, 0)),
            pl.BlockSpec((1, HD), lambda i: (0, 0)),
            pl.BlockSpec((1, 1, BM), lambda i: (i, 0, 0)),
            pl.BlockSpec((HD, 2), lambda i: (0, 0)),
            pl.BlockSpec((1, 2), lambda i: (0, 0)),
        ],
        out_specs=pl.BlockSpec((G, 2), lambda i: (0, 0)),
        out_shape=_f32(G, 2),
        scratch_shapes=[pltpu.VMEM((G, HD), jnp.float32),
                        pltpu.VMEM((G, 1), jnp.float32)],
    )(a2, den2, b2, batch2d, Wl, bl)


def kernel(x, edge_index, batch, W1, as1, ad1, b1, W2, as2, ad2, b2, Wl, bl):
    f32 = jnp.float32
    loop = jnp.arange(N, dtype=jnp.int32)
    src = jnp.concatenate(
        [edge_index[0].astype(jnp.int32), loop,
         jnp.zeros((ETP - ET,), jnp.int32)])
    dst = jnp.concatenate(
        [edge_index[1].astype(jnp.int32), loop,
         jnp.full((ETP - ET,), N, jnp.int32)])
    x_pad = jnp.pad(x, ((0, NP - N), (0, 0)))
    batch2d = jnp.pad(batch.astype(jnp.int32), (0, NP - N),
                      constant_values=G).reshape(NBLK, 1, BM)
    z128 = jnp.zeros((128, 128), f32)

    h1c, asrc1, adst1, c1 = _tc1(x_pad, W1, as1, ad1)
    ex1, den1 = _sc_b(H, src, dst, asrc1.reshape(-1), adst1.reshape(-1),
                      c1, z128)
    a1 = _sc_c(F1 // 128, H, src, dst, ex1, h1c.reshape(-1, 128), z128)

    W2c = W2.reshape(F1 // 128, 128, HD)
    b1c = b1.reshape(F1 // 128, 128)
    h2c, asrc2, adst2, c2 = _tc2(a1.reshape(F1 // 128, NP, 128),
                                 den1.reshape(NC, NP, 16), W2c, b1c,
                                 as2, ad2)
    ex2, den2 = _sc_b(1, src, dst, asrc2.reshape(-1), adst2.reshape(-1),
                      c2, z128)
    a2 = _sc_c(2, 1, src, dst, ex2, h2c.reshape(-1, 128), z128)

    den2r = den2.reshape(NC, NP, 16)
    return _tc3(a2.reshape(2, NP, 128), den2r,
                b2.reshape(1, HD), batch2d, Wl, bl.reshape(1, 2))
